# baseline (speedup 1.0000x reference)
import functools

import jax
import jax.numpy as jnp
from jax import lax
from jax.experimental import pallas as pl
from jax.experimental.pallas import tpu as pltpu
from jax.experimental.pallas import tpu_sc as plsc

N, E, DF, DA, H = 10000, 160000, 128, 4, 64
NC, NS, L = 2, 16, 16
NW = NC * NS
IDXW = 128
CP = 80
EPAD = NS * CP * IDXW
C0A, C1A = 48, 0
C0B, C1B = 32, 0
NPAD = 10240
DUMMY = NPAD - 8
HP = 128
NP2 = NPAD // 2
STRIPE2 = NP2 // NS
SBLK = 512
SBPW = 10


def _vmesh():
    return plsc.VectorSubcoreMesh(core_axis_name="c", subcore_axis_name="s")


def _make_gather(b0, n0, b1, n1):
    @functools.partial(
        pl.kernel,
        out_type=(jax.ShapeDtypeStruct((EPAD, HP), jnp.float32),
                  jax.ShapeDtypeStruct((EPAD, HP), jnp.float32)),
        mesh=_vmesh(),
        scratch_types=[pltpu.VMEM((max(n0, n1), IDXW), jnp.int32),
                       pltpu.VMEM((max(n0, n1), IDXW), jnp.int32),
                       pltpu.VMEM((IDXW, HP), jnp.float32),
                       pltpu.VMEM((IDXW, HP), jnp.float32),
                       pltpu.VMEM((IDXW, HP), jnp.float32),
                       pltpu.VMEM((IDXW, HP), jnp.float32),
                       pltpu.SemaphoreType.DMA,
                       pltpu.SemaphoreType.DMA,
                       pltpu.SemaphoreType.DMA],
    )
    def gather(nodes_hbm, srcm_hbm, dstm_hbm, gsrc_hbm, gdst_hbm,
               idxs_v, idxd_v, sb0, sb1, db0, db1, gsem_s, gsem_d, wsem):
        cid = lax.axis_index("c")
        sid = lax.axis_index("s")
        sb = (sb0, sb1)
        db = (db0, db1)

        def run(row0, nchunks):
            e0 = row0 * IDXW
            pltpu.sync_copy(srcm_hbm.at[pl.ds(row0, nchunks)],
                            idxs_v.at[pl.ds(0, nchunks)])
            pltpu.sync_copy(dstm_hbm.at[pl.ds(row0, nchunks)],
                            idxd_v.at[pl.ds(0, nchunks)])
            descs = {}

            def fire(c):
                p = c & 1
                w_prev = descs.pop(("w", c - 2), None)
                if w_prev is not None:
                    w_prev.wait()
                    descs.pop(("w2", c - 2)).wait()
                descs[("gs", c)] = pltpu.async_copy(
                    nodes_hbm.at[idxs_v.at[c]], sb[p], gsem_s)
                descs[("gd", c)] = pltpu.async_copy(
                    nodes_hbm.at[idxd_v.at[c]], db[p], gsem_d)

            fire(0)
            fire(1)
            for c in range(nchunks):
                p = c & 1
                descs.pop(("gs", c)).wait()
                descs.pop(("gd", c)).wait()
                rows = pl.ds(e0 + c * IDXW, IDXW)
                descs[("w", c)] = pltpu.async_copy(
                    sb[p], gsrc_hbm.at[rows], wsem)
                descs[("w2", c)] = pltpu.async_copy(
                    db[p], gdst_hbm.at[rows], wsem)
                if c + 2 < nchunks:
                    fire(c + 2)
            for c in (nchunks - 2, nchunks - 1):
                descs.pop(("w", c)).wait()
                descs.pop(("w2", c)).wait()

        @pl.when(cid == 0)
        def _():
            run(sid * CP + b0, n0)

        if n1:
            @pl.when(cid == 1)
            def _():
                run(sid * CP + 64 + b1, n1)

    return gather


_sc_gather_a = _make_gather(0, C0A, 0, C1A)
_sc_gather_b = _make_gather(C0A, C0B, C1A, C1B)


@functools.partial(
    pl.kernel,
    out_type=jax.ShapeDtypeStruct((NC, NP2, HP), jnp.float32),
    mesh=_vmesh(),
    scratch_types=[pltpu.VMEM((4, IDXW), jnp.int32),
                   pltpu.VMEM((SBLK, HP), jnp.float32),
                   pltpu.VMEM((IDXW, HP), jnp.float32),
                   pltpu.VMEM_SHARED((NP2, HP), jnp.float32),
                   pltpu.SemaphoreType.DMA],
)
def _sc_scatter(m2a_hbm, m2b_hbm, dstm_hbm, out_hbm,
                idx_v, rows_v, zbuf, shared, sem):
    cid = lax.axis_index("c")
    sid = lax.axis_index("s")

    def zrow(i, carry):
        for j in range(HP // L):
            zbuf[i, pl.ds(j * L, L)] = jnp.zeros((L,), jnp.float32)
        return carry

    lax.fori_loop(0, IDXW, zrow, 0)
    for k in range(STRIPE2 // 64):
        pltpu.sync_copy(zbuf.at[pl.ds(0, 64)],
                        shared.at[pl.ds(sid * STRIPE2 + k * 64, 64)])
    plsc.subcore_barrier()

    for bb in range(SBPW):
        for cc in range(NC):
            b = bb + cc * SBPW
            u = b // 2
            m2_hbm = m2a_hbm if u in (0, 1, 2, 3, 4, 5) else m2b_hbm
            row0 = sid * CP + b * 4
            e0 = row0 * IDXW
            with_core = pl.when(cid == cc)

            @with_core
            def _(m2_hbm=m2_hbm, row0=row0, e0=e0):
                pltpu.sync_copy(dstm_hbm.at[pl.ds(row0, 4)], idx_v)
                pltpu.sync_copy(m2_hbm.at[pl.ds(e0, SBLK)], rows_v)
                for j in range(4):
                    pltpu.sync_copy(rows_v.at[pl.ds(j * IDXW, IDXW)],
                                    shared.at[idx_v.at[j]], add=True)

    plsc.subcore_barrier()
    pltpu.sync_copy(shared.at[pl.ds(sid * STRIPE2, STRIPE2)],
                    out_hbm.at[cid, pl.ds(sid * STRIPE2, STRIPE2)])


def _combine(y, attrs):
    acc = attrs[:, 0:1] * y[:, 0:H]
    for a in range(1, DA):
        acc = acc + attrs[:, a:a + 1] * y[:, a * H:(a + 1) * H]
    return acc


def _pad_cols(v):
    return jnp.concatenate([v, jnp.zeros_like(v)], axis=1)


def _embed_body(x_ref, na_ref, w_ref, b_ref, o_ref):
    y = jnp.dot(x_ref[...], w_ref[...], preferred_element_type=jnp.float32)
    o_ref[...] = _pad_cols(_combine(y, na_ref[...]) + b_ref[...])


def _edge_body(gs_ref, gd_ref, ea_ref, w1s_ref, w1d_ref, b1_ref,
               w2_ref, b2_ref, o_ref):
    ea = ea_ref[...]
    par = ea[:, DA:DA + 1]
    gs = gs_ref[:, :H]
    gd = gd_ref[:, :H]
    y = (jnp.dot(gs, w1s_ref[...], preferred_element_type=jnp.float32)
         + jnp.dot(gd, w1d_ref[...], preferred_element_type=jnp.float32))
    t = jax.nn.gelu(_combine(y, ea) + b1_ref[...])
    y2 = jnp.dot(t, w2_ref[...], preferred_element_type=jnp.float32)
    m2 = _combine(y2, ea) + b2_ref[...]
    o_ref[...] = jnp.concatenate([m2 * (1.0 - par), m2 * par], axis=1)


def _node_body(n_ref, agg_ref, na_ref, wt_ref, wb_ref, bn_ref, o_ref):
    nodes = n_ref[:, :H]
    agg = agg_ref[0] + agg_ref[1]
    y = (jnp.dot(nodes, wt_ref[...], preferred_element_type=jnp.float32)
         + jnp.dot(agg, wb_ref[...], preferred_element_type=jnp.float32))
    o_ref[...] = _pad_cols(_combine(y, na_ref[...]) + bn_ref[...] + nodes)


_NB = 1000

_tc_embed = pl.pallas_call(
    _embed_body,
    grid=(N // _NB,),
    in_specs=[pl.BlockSpec((_NB, DF), lambda i: (i, 0)),
              pl.BlockSpec((_NB, DA), lambda i: (i, 0)),
              pl.BlockSpec((DF, DA * H), lambda i: (0, 0)),
              pl.BlockSpec((1, H), lambda i: (0, 0))],
    out_specs=pl.BlockSpec((_NB, HP), lambda i: (i, 0)),
    out_shape=jax.ShapeDtypeStruct((N, HP), jnp.float32),
)

_EBLK = 1024


def _make_edge(units):
    unit_fn, nblk = units

    def imap(i):
        return ((i // nblk) * (CP * IDXW // _EBLK) + unit_fn(i % nblk), 0)

    return pl.pallas_call(
        _edge_body,
        grid=(NS * nblk,),
        in_specs=[pl.BlockSpec((_EBLK, HP), imap),
                  pl.BlockSpec((_EBLK, HP), imap),
                  pl.BlockSpec((_EBLK, 8), imap),
                  pl.BlockSpec((H, DA * H), lambda i: (0, 0)),
                  pl.BlockSpec((H, DA * H), lambda i: (0, 0)),
                  pl.BlockSpec((1, H), lambda i: (0, 0)),
                  pl.BlockSpec((H, DA * H), lambda i: (0, 0)),
                  pl.BlockSpec((1, H), lambda i: (0, 0))],
        out_specs=pl.BlockSpec((_EBLK, HP), imap),
        out_shape=jax.ShapeDtypeStruct((EPAD, HP), jnp.float32),
    )


_tc_edge_a = _make_edge((lambda j: j, 6))
_tc_edge_b = _make_edge((lambda j: 6 + j, 4))

_tc_node = pl.pallas_call(
    _node_body,
    grid=(N // _NB,),
    in_specs=[pl.BlockSpec((_NB, HP), lambda i: (i, 0)),
              pl.BlockSpec((NC, _NB, H), lambda i: (0, i, 0)),
              pl.BlockSpec((_NB, DA), lambda i: (i, 0)),
              pl.BlockSpec((H, DA * H), lambda i: (0, 0)),
              pl.BlockSpec((H, DA * H), lambda i: (0, 0)),
              pl.BlockSpec((1, H), lambda i: (0, 0))],
    out_specs=pl.BlockSpec((_NB, HP), lambda i: (i, 0)),
    out_shape=jax.ShapeDtypeStruct((N, HP), jnp.float32),
)


def _rw(w):
    return w.reshape(-1, DA, H).reshape(-1, DA * H)


def kernel(x, edge_index, steerable_node_attrs, steerable_edge_attrs,
           W_embed, b_embed, We1_0, be1_0, We2_0, be2_0, Wn_0, bn_0,
           We1_1, be1_1, We2_1, be2_1, Wn_1, bn_1,
           We1_2, be1_2, We2_2, be2_2, Wn_2, bn_2):
    src = edge_index[0]
    dst = edge_index[1]
    pad = EPAD - E
    zi = jnp.zeros((pad,), jnp.int32)
    srcm = jnp.concatenate([src, zi]).reshape(EPAD // IDXW, IDXW)
    dstg = jnp.concatenate([dst, zi]).reshape(EPAD // IDXW, IDXW)
    dstp = jnp.concatenate([dst, jnp.full((pad,), DUMMY, jnp.int32)])
    dsts = (dstp >> 1).reshape(EPAD // IDXW, IDXW)
    parity = (dstp & 1).astype(jnp.float32).reshape(EPAD, 1)
    eap = jnp.concatenate(
        [jnp.concatenate(
            [steerable_edge_attrs, jnp.zeros((pad, DA), jnp.float32)], axis=0),
         parity, jnp.zeros((EPAD, 3), jnp.float32)], axis=1)
    na = steerable_node_attrs

    nodes = _tc_embed(x, na, _rw(W_embed), b_embed.reshape(1, H))
    for (We1, be1, We2, be2, Wn, bn) in (
            (We1_0, be1_0, We2_0, be2_0, Wn_0, bn_0),
            (We1_1, be1_1, We2_1, be2_1, Wn_1, bn_1),
            (We1_2, be1_2, We2_2, be2_2, Wn_2, bn_2)):
        W1 = _rw(We1)
        W2 = _rw(We2)
        Wn_r = _rw(Wn)
        b1 = be1.reshape(1, H)
        b2 = be2.reshape(1, H)
        gsA, gdA = _sc_gather_a(nodes, srcm, dstg)
        gsB, gdB = _sc_gather_b(nodes, srcm, dstg)
        m2A = _tc_edge_a(gsA, gdA, eap, W1[:H], W1[H:], b1, W2, b2)
        m2B = _tc_edge_b(gsB, gdB, eap, W1[:H], W1[H:], b1, W2, b2)
        aggp = _sc_scatter(m2A, m2B, dsts).reshape(NC, NPAD, H)
        nodes = _tc_node(nodes, aggp, na, Wn_r[:H], Wn_r[H:],
                         bn.reshape(1, H))
    return nodes[:, :H]


# speedup vs baseline: 1.1371x; 1.1371x over previous
import functools

import jax
import jax.numpy as jnp
from jax import lax
from jax.experimental import pallas as pl
from jax.experimental.pallas import tpu as pltpu
from jax.experimental.pallas import tpu_sc as plsc

N, E, DF, DA, H = 10000, 160000, 128, 4, 64
NC, NS, L = 2, 16, 16
NW = NC * NS
IDXW = 128
CP = 80
EPAD = NS * CP * IDXW
C0A, C1A = 40, 8
C0B, C1B = 24, 8
NPAD = 10240
DUMMY = NPAD - 8
HP = 128
NP2 = NPAD // 2
STRIPE2 = NP2 // NS
SBLK = 512
SBPW = 10


def _vmesh():
    return plsc.VectorSubcoreMesh(core_axis_name="c", subcore_axis_name="s")


def _make_gather(b0, n0, b1, n1):
    @functools.partial(
        pl.kernel,
        out_type=(jax.ShapeDtypeStruct((EPAD, HP), jnp.float32),
                  jax.ShapeDtypeStruct((EPAD, HP), jnp.float32)),
        mesh=_vmesh(),
        scratch_types=[pltpu.VMEM((max(n0, n1), IDXW), jnp.int32),
                       pltpu.VMEM((max(n0, n1), IDXW), jnp.int32),
                       pltpu.VMEM((IDXW, HP), jnp.float32),
                       pltpu.VMEM((IDXW, HP), jnp.float32),
                       pltpu.VMEM((IDXW, HP), jnp.float32),
                       pltpu.VMEM((IDXW, HP), jnp.float32),
                       pltpu.SemaphoreType.DMA,
                       pltpu.SemaphoreType.DMA,
                       pltpu.SemaphoreType.DMA],
    )
    def gather(nodes_hbm, srcm_hbm, dstm_hbm, gsrc_hbm, gdst_hbm,
               idxs_v, idxd_v, sb0, sb1, db0, db1, gsem_s, gsem_d, wsem):
        cid = lax.axis_index("c")
        sid = lax.axis_index("s")
        sb = (sb0, sb1)
        db = (db0, db1)

        def run(row0, nchunks):
            e0 = row0 * IDXW
            pltpu.sync_copy(srcm_hbm.at[pl.ds(row0, nchunks)],
                            idxs_v.at[pl.ds(0, nchunks)])
            pltpu.sync_copy(dstm_hbm.at[pl.ds(row0, nchunks)],
                            idxd_v.at[pl.ds(0, nchunks)])
            descs = {}

            def fire(c):
                p = c & 1
                w_prev = descs.pop(("w", c - 2), None)
                if w_prev is not None:
                    w_prev.wait()
                    descs.pop(("w2", c - 2)).wait()
                descs[("gs", c)] = pltpu.async_copy(
                    nodes_hbm.at[idxs_v.at[c]], sb[p], gsem_s)
                descs[("gd", c)] = pltpu.async_copy(
                    nodes_hbm.at[idxd_v.at[c]], db[p], gsem_d)

            fire(0)
            fire(1)
            for c in range(nchunks):
                p = c & 1
                descs.pop(("gs", c)).wait()
                descs.pop(("gd", c)).wait()
                rows = pl.ds(e0 + c * IDXW, IDXW)
                descs[("w", c)] = pltpu.async_copy(
                    sb[p], gsrc_hbm.at[rows], wsem)
                descs[("w2", c)] = pltpu.async_copy(
                    db[p], gdst_hbm.at[rows], wsem)
                if c + 2 < nchunks:
                    fire(c + 2)
            for c in (nchunks - 2, nchunks - 1):
                descs.pop(("w", c)).wait()
                descs.pop(("w2", c)).wait()

        @pl.when(cid == 0)
        def _():
            run(sid * CP + b0, n0)

        if n1:
            @pl.when(cid == 1)
            def _():
                run(sid * CP + 64 + b1, n1)

    return gather


_sc_gather_a = _make_gather(0, C0A, 0, C1A)
_sc_gather_b = _make_gather(C0A, C0B, C1A, C1B)


@functools.partial(
    pl.kernel,
    out_type=jax.ShapeDtypeStruct((NC, NP2, HP), jnp.float32),
    mesh=_vmesh(),
    scratch_types=[pltpu.VMEM((4, IDXW), jnp.int32),
                   pltpu.VMEM((SBLK, HP), jnp.float32),
                   pltpu.VMEM((IDXW, HP), jnp.float32),
                   pltpu.VMEM_SHARED((NP2, HP), jnp.float32),
                   pltpu.SemaphoreType.DMA],
)
def _sc_scatter(m2a_hbm, m2b_hbm, dstm_hbm, out_hbm,
                idx_v, rows_v, zbuf, shared, sem):
    cid = lax.axis_index("c")
    sid = lax.axis_index("s")

    def zrow(i, carry):
        for j in range(HP // L):
            zbuf[i, pl.ds(j * L, L)] = jnp.zeros((L,), jnp.float32)
        return carry

    lax.fori_loop(0, IDXW, zrow, 0)
    for k in range(STRIPE2 // 64):
        pltpu.sync_copy(zbuf.at[pl.ds(0, 64)],
                        shared.at[pl.ds(sid * STRIPE2 + k * 64, 64)])
    plsc.subcore_barrier()

    for bb in range(SBPW):
        for cc in range(NC):
            b = bb + cc * SBPW
            u = b // 2
            m2_hbm = m2a_hbm if u in (0, 1, 2, 3, 4, 8) else m2b_hbm
            row0 = sid * CP + b * 4
            e0 = row0 * IDXW
            with_core = pl.when(cid == cc)

            @with_core
            def _(m2_hbm=m2_hbm, row0=row0, e0=e0):
                pltpu.sync_copy(dstm_hbm.at[pl.ds(row0, 4)], idx_v)
                pltpu.sync_copy(m2_hbm.at[pl.ds(e0, SBLK)], rows_v)
                for j in range(4):
                    pltpu.sync_copy(rows_v.at[pl.ds(j * IDXW, IDXW)],
                                    shared.at[idx_v.at[j]], add=True)

    plsc.subcore_barrier()
    pltpu.sync_copy(shared.at[pl.ds(sid * STRIPE2, STRIPE2)],
                    out_hbm.at[cid, pl.ds(sid * STRIPE2, STRIPE2)])


def _combine(y, attrs):
    acc = attrs[:, 0:1] * y[:, 0:H]
    for a in range(1, DA):
        acc = acc + attrs[:, a:a + 1] * y[:, a * H:(a + 1) * H]
    return acc


def _pad_cols(v):
    return jnp.concatenate([v, jnp.zeros_like(v)], axis=1)


def _embed_body(x_ref, na_ref, w_ref, b_ref, o_ref):
    y = jnp.dot(x_ref[...], w_ref[...], preferred_element_type=jnp.float32)
    o_ref[...] = _pad_cols(_combine(y, na_ref[...]) + b_ref[...])


def _edge_body(gs_ref, gd_ref, ea_ref, w1s_ref, w1d_ref, b1_ref,
               w2_ref, b2_ref, o_ref):
    ea = ea_ref[...]
    par = ea[:, DA:DA + 1]
    gs = gs_ref[:, :H]
    gd = gd_ref[:, :H]
    y = (jnp.dot(gs, w1s_ref[...], preferred_element_type=jnp.float32)
         + jnp.dot(gd, w1d_ref[...], preferred_element_type=jnp.float32))
    t = jax.nn.gelu(_combine(y, ea) + b1_ref[...])
    y2 = jnp.dot(t, w2_ref[...], preferred_element_type=jnp.float32)
    m2 = _combine(y2, ea) + b2_ref[...]
    o_ref[...] = jnp.concatenate([m2 * (1.0 - par), m2 * par], axis=1)


def _node_body(n_ref, agg_ref, na_ref, wt_ref, wb_ref, bn_ref, o_ref):
    nodes = n_ref[:, :H]
    agg = agg_ref[0] + agg_ref[1]
    y = (jnp.dot(nodes, wt_ref[...], preferred_element_type=jnp.float32)
         + jnp.dot(agg, wb_ref[...], preferred_element_type=jnp.float32))
    o_ref[...] = _pad_cols(_combine(y, na_ref[...]) + bn_ref[...] + nodes)


_NB = 1000

_tc_embed = pl.pallas_call(
    _embed_body,
    grid=(N // _NB,),
    in_specs=[pl.BlockSpec((_NB, DF), lambda i: (i, 0)),
              pl.BlockSpec((_NB, DA), lambda i: (i, 0)),
              pl.BlockSpec((DF, DA * H), lambda i: (0, 0)),
              pl.BlockSpec((1, H), lambda i: (0, 0))],
    out_specs=pl.BlockSpec((_NB, HP), lambda i: (i, 0)),
    out_shape=jax.ShapeDtypeStruct((N, HP), jnp.float32),
)

_EBLK = 1024


def _make_edge(units):
    unit_fn, nblk = units

    def imap(i):
        return ((i // nblk) * (CP * IDXW // _EBLK) + unit_fn(i % nblk), 0)

    return pl.pallas_call(
        _edge_body,
        grid=(NS * nblk,),
        in_specs=[pl.BlockSpec((_EBLK, HP), imap),
                  pl.BlockSpec((_EBLK, HP), imap),
                  pl.BlockSpec((_EBLK, 8), imap),
                  pl.BlockSpec((H, DA * H), lambda i: (0, 0)),
                  pl.BlockSpec((H, DA * H), lambda i: (0, 0)),
                  pl.BlockSpec((1, H), lambda i: (0, 0)),
                  pl.BlockSpec((H, DA * H), lambda i: (0, 0)),
                  pl.BlockSpec((1, H), lambda i: (0, 0))],
        out_specs=pl.BlockSpec((_EBLK, HP), imap),
        out_shape=jax.ShapeDtypeStruct((EPAD, HP), jnp.float32),
    )


_tc_edge_a = _make_edge((lambda j: jnp.where(j < 5, j, 8), 6))
_tc_edge_b = _make_edge((lambda j: jnp.where(j < 3, 5 + j, 9), 4))

_tc_node = pl.pallas_call(
    _node_body,
    grid=(N // _NB,),
    in_specs=[pl.BlockSpec((_NB, HP), lambda i: (i, 0)),
              pl.BlockSpec((NC, _NB, H), lambda i: (0, i, 0)),
              pl.BlockSpec((_NB, DA), lambda i: (i, 0)),
              pl.BlockSpec((H, DA * H), lambda i: (0, 0)),
              pl.BlockSpec((H, DA * H), lambda i: (0, 0)),
              pl.BlockSpec((1, H), lambda i: (0, 0))],
    out_specs=pl.BlockSpec((_NB, HP), lambda i: (i, 0)),
    out_shape=jax.ShapeDtypeStruct((N, HP), jnp.float32),
)


def _rw(w):
    return w.reshape(-1, DA, H).reshape(-1, DA * H)


def kernel(x, edge_index, steerable_node_attrs, steerable_edge_attrs,
           W_embed, b_embed, We1_0, be1_0, We2_0, be2_0, Wn_0, bn_0,
           We1_1, be1_1, We2_1, be2_1, Wn_1, bn_1,
           We1_2, be1_2, We2_2, be2_2, Wn_2, bn_2):
    src = edge_index[0]
    dst = edge_index[1]
    pad = EPAD - E
    zi = jnp.zeros((pad,), jnp.int32)
    srcm = jnp.concatenate([src, zi]).reshape(EPAD // IDXW, IDXW)
    dstg = jnp.concatenate([dst, zi]).reshape(EPAD // IDXW, IDXW)
    dstp = jnp.concatenate([dst, jnp.full((pad,), DUMMY, jnp.int32)])
    dsts = (dstp >> 1).reshape(EPAD // IDXW, IDXW)
    parity = (dstp & 1).astype(jnp.float32).reshape(EPAD, 1)
    eap = jnp.concatenate(
        [jnp.concatenate(
            [steerable_edge_attrs, jnp.zeros((pad, DA), jnp.float32)], axis=0),
         parity, jnp.zeros((EPAD, 3), jnp.float32)], axis=1)
    na = steerable_node_attrs

    nodes = _tc_embed(x, na, _rw(W_embed), b_embed.reshape(1, H))
    for (We1, be1, We2, be2, Wn, bn) in (
            (We1_0, be1_0, We2_0, be2_0, Wn_0, bn_0),
            (We1_1, be1_1, We2_1, be2_1, Wn_1, bn_1),
            (We1_2, be1_2, We2_2, be2_2, Wn_2, bn_2)):
        W1 = _rw(We1)
        W2 = _rw(We2)
        Wn_r = _rw(Wn)
        b1 = be1.reshape(1, H)
        b2 = be2.reshape(1, H)
        gsA, gdA = _sc_gather_a(nodes, srcm, dstg)
        gsB, gdB = _sc_gather_b(nodes, srcm, dstg)
        m2A = _tc_edge_a(gsA, gdA, eap, W1[:H], W1[H:], b1, W2, b2)
        m2B = _tc_edge_b(gsB, gdB, eap, W1[:H], W1[H:], b1, W2, b2)
        aggp = _sc_scatter(m2A, m2B, dsts).reshape(NC, NPAD, H)
        nodes = _tc_node(nodes, aggp, na, Wn_r[:H], Wn_r[H:],
                         bn.reshape(1, H))
    return nodes[:, :H]


# speedup vs baseline: 1.1499x; 1.0113x over previous
import functools

import jax
import jax.numpy as jnp
from jax import lax
from jax.experimental import pallas as pl
from jax.experimental.pallas import tpu as pltpu
from jax.experimental.pallas import tpu_sc as plsc

N, E, DF, DA, H = 10000, 160000, 128, 4, 64
NC, NS, L = 2, 16, 16
NW = NC * NS
IDXW = 128
CP = 80
EPAD = NS * CP * IDXW
C0A, C1A = 32, 8
C0B, C1B = 24, 16
NPAD = 10240
DUMMY = NPAD - 8
HP = 128
NP2 = NPAD // 2
STRIPE2 = NP2 // NS
SBLK = 512
SBPW = 10


def _vmesh():
    return plsc.VectorSubcoreMesh(core_axis_name="c", subcore_axis_name="s")


def _make_gather(b0, n0, b1, n1):
    @functools.partial(
        pl.kernel,
        out_type=(jax.ShapeDtypeStruct((EPAD, HP), jnp.float32),
                  jax.ShapeDtypeStruct((EPAD, HP), jnp.float32)),
        mesh=_vmesh(),
        scratch_types=[pltpu.VMEM((max(n0, n1), IDXW), jnp.int32),
                       pltpu.VMEM((max(n0, n1), IDXW), jnp.int32),
                       pltpu.VMEM((IDXW, HP), jnp.float32),
                       pltpu.VMEM((IDXW, HP), jnp.float32),
                       pltpu.VMEM((IDXW, HP), jnp.float32),
                       pltpu.VMEM((IDXW, HP), jnp.float32),
                       pltpu.SemaphoreType.DMA,
                       pltpu.SemaphoreType.DMA,
                       pltpu.SemaphoreType.DMA],
    )
    def gather(nodes_hbm, srcm_hbm, dstm_hbm, gsrc_hbm, gdst_hbm,
               idxs_v, idxd_v, sb0, sb1, db0, db1, gsem_s, gsem_d, wsem):
        cid = lax.axis_index("c")
        sid = lax.axis_index("s")
        sb = (sb0, sb1)
        db = (db0, db1)

        def run(row0, nchunks):
            e0 = row0 * IDXW
            pltpu.sync_copy(srcm_hbm.at[pl.ds(row0, nchunks)],
                            idxs_v.at[pl.ds(0, nchunks)])
            pltpu.sync_copy(dstm_hbm.at[pl.ds(row0, nchunks)],
                            idxd_v.at[pl.ds(0, nchunks)])
            descs = {}

            def fire(c):
                p = c & 1
                w_prev = descs.pop(("w", c - 2), None)
                if w_prev is not None:
                    w_prev.wait()
                    descs.pop(("w2", c - 2)).wait()
                descs[("gs", c)] = pltpu.async_copy(
                    nodes_hbm.at[idxs_v.at[c]], sb[p], gsem_s)
                descs[("gd", c)] = pltpu.async_copy(
                    nodes_hbm.at[idxd_v.at[c]], db[p], gsem_d)

            fire(0)
            fire(1)
            for c in range(nchunks):
                p = c & 1
                descs.pop(("gs", c)).wait()
                descs.pop(("gd", c)).wait()
                rows = pl.ds(e0 + c * IDXW, IDXW)
                descs[("w", c)] = pltpu.async_copy(
                    sb[p], gsrc_hbm.at[rows], wsem)
                descs[("w2", c)] = pltpu.async_copy(
                    db[p], gdst_hbm.at[rows], wsem)
                if c + 2 < nchunks:
                    fire(c + 2)
            for c in (nchunks - 2, nchunks - 1):
                descs.pop(("w", c)).wait()
                descs.pop(("w2", c)).wait()

        @pl.when(cid == 0)
        def _():
            run(sid * CP + b0, n0)

        if n1:
            @pl.when(cid == 1)
            def _():
                run(sid * CP + 56 + b1, n1)

    return gather


_sc_gather_a = _make_gather(0, C0A, 0, C1A)
_sc_gather_b = _make_gather(C0A, C0B, C1A, C1B)


@functools.partial(
    pl.kernel,
    out_type=jax.ShapeDtypeStruct((NC, NP2, HP), jnp.float32),
    mesh=_vmesh(),
    scratch_types=[pltpu.VMEM((4, IDXW), jnp.int32),
                   pltpu.VMEM((SBLK, HP), jnp.float32),
                   pltpu.VMEM((IDXW, HP), jnp.float32),
                   pltpu.VMEM_SHARED((NP2, HP), jnp.float32),
                   pltpu.SemaphoreType.DMA],
)
def _sc_scatter(m2a_hbm, m2b_hbm, dstm_hbm, out_hbm,
                idx_v, rows_v, zbuf, shared, sem):
    cid = lax.axis_index("c")
    sid = lax.axis_index("s")

    def zrow(i, carry):
        for j in range(HP // L):
            zbuf[i, pl.ds(j * L, L)] = jnp.zeros((L,), jnp.float32)
        return carry

    lax.fori_loop(0, IDXW, zrow, 0)
    for k in range(STRIPE2 // 64):
        pltpu.sync_copy(zbuf.at[pl.ds(0, 64)],
                        shared.at[pl.ds(sid * STRIPE2 + k * 64, 64)])
    plsc.subcore_barrier()

    for bb in range(SBPW):
        for cc in range(NC):
            b = bb + cc * SBPW
            u = b // 2
            m2_hbm = m2a_hbm if u in (0, 1, 2, 3, 7) else m2b_hbm
            row0 = sid * CP + b * 4
            e0 = row0 * IDXW
            with_core = pl.when(cid == cc)

            @with_core
            def _(m2_hbm=m2_hbm, row0=row0, e0=e0):
                pltpu.sync_copy(dstm_hbm.at[pl.ds(row0, 4)], idx_v)
                pltpu.sync_copy(m2_hbm.at[pl.ds(e0, SBLK)], rows_v)
                for j in range(4):
                    pltpu.sync_copy(rows_v.at[pl.ds(j * IDXW, IDXW)],
                                    shared.at[idx_v.at[j]], add=True)

    plsc.subcore_barrier()
    pltpu.sync_copy(shared.at[pl.ds(sid * STRIPE2, STRIPE2)],
                    out_hbm.at[cid, pl.ds(sid * STRIPE2, STRIPE2)])


def _combine(y, attrs):
    acc = attrs[:, 0:1] * y[:, 0:H]
    for a in range(1, DA):
        acc = acc + attrs[:, a:a + 1] * y[:, a * H:(a + 1) * H]
    return acc


def _pad_cols(v):
    return jnp.concatenate([v, jnp.zeros_like(v)], axis=1)


def _embed_body(x_ref, na_ref, w_ref, b_ref, o_ref):
    y = jnp.dot(x_ref[...], w_ref[...], preferred_element_type=jnp.float32)
    o_ref[...] = _pad_cols(_combine(y, na_ref[...]) + b_ref[...])


def _edge_body(gs_ref, gd_ref, ea_ref, w1s_ref, w1d_ref, b1_ref,
               w2_ref, b2_ref, o_ref):
    ea = ea_ref[...]
    par = ea[:, DA:DA + 1]
    gs = gs_ref[:, :H]
    gd = gd_ref[:, :H]
    y = (jnp.dot(gs, w1s_ref[...], preferred_element_type=jnp.float32)
         + jnp.dot(gd, w1d_ref[...], preferred_element_type=jnp.float32))
    t = jax.nn.gelu(_combine(y, ea) + b1_ref[...])
    y2 = jnp.dot(t, w2_ref[...], preferred_element_type=jnp.float32)
    m2 = _combine(y2, ea) + b2_ref[...]
    o_ref[...] = jnp.concatenate([m2 * (1.0 - par), m2 * par], axis=1)


def _node_body(n_ref, agg_ref, na_ref, wt_ref, wb_ref, bn_ref, o_ref):
    nodes = n_ref[:, :H]
    agg = agg_ref[0] + agg_ref[1]
    y = (jnp.dot(nodes, wt_ref[...], preferred_element_type=jnp.float32)
         + jnp.dot(agg, wb_ref[...], preferred_element_type=jnp.float32))
    o_ref[...] = _pad_cols(_combine(y, na_ref[...]) + bn_ref[...] + nodes)


_NB = 1000

_tc_embed = pl.pallas_call(
    _embed_body,
    grid=(N // _NB,),
    in_specs=[pl.BlockSpec((_NB, DF), lambda i: (i, 0)),
              pl.BlockSpec((_NB, DA), lambda i: (i, 0)),
              pl.BlockSpec((DF, DA * H), lambda i: (0, 0)),
              pl.BlockSpec((1, H), lambda i: (0, 0))],
    out_specs=pl.BlockSpec((_NB, HP), lambda i: (i, 0)),
    out_shape=jax.ShapeDtypeStruct((N, HP), jnp.float32),
)

_EBLK = 1024


def _make_edge(units):
    unit_fn, nblk = units

    def imap(i):
        return ((i // nblk) * (CP * IDXW // _EBLK) + unit_fn(i % nblk), 0)

    return pl.pallas_call(
        _edge_body,
        grid=(NS * nblk,),
        in_specs=[pl.BlockSpec((_EBLK, HP), imap),
                  pl.BlockSpec((_EBLK, HP), imap),
                  pl.BlockSpec((_EBLK, 8), imap),
                  pl.BlockSpec((H, DA * H), lambda i: (0, 0)),
                  pl.BlockSpec((H, DA * H), lambda i: (0, 0)),
                  pl.BlockSpec((1, H), lambda i: (0, 0)),
                  pl.BlockSpec((H, DA * H), lambda i: (0, 0)),
                  pl.BlockSpec((1, H), lambda i: (0, 0))],
        out_specs=pl.BlockSpec((_EBLK, HP), imap),
        out_shape=jax.ShapeDtypeStruct((EPAD, HP), jnp.float32),
    )


_tc_edge_a = _make_edge((lambda j: jnp.where(j < 4, j, 7), 5))
_tc_edge_b = _make_edge((lambda j: jnp.where(j < 3, 4 + j, 5 + j), 5))

_tc_node = pl.pallas_call(
    _node_body,
    grid=(N // _NB,),
    in_specs=[pl.BlockSpec((_NB, HP), lambda i: (i, 0)),
              pl.BlockSpec((NC, _NB, H), lambda i: (0, i, 0)),
              pl.BlockSpec((_NB, DA), lambda i: (i, 0)),
              pl.BlockSpec((H, DA * H), lambda i: (0, 0)),
              pl.BlockSpec((H, DA * H), lambda i: (0, 0)),
              pl.BlockSpec((1, H), lambda i: (0, 0))],
    out_specs=pl.BlockSpec((_NB, HP), lambda i: (i, 0)),
    out_shape=jax.ShapeDtypeStruct((N, HP), jnp.float32),
)


def _rw(w):
    return w.reshape(-1, DA, H).reshape(-1, DA * H)


def kernel(x, edge_index, steerable_node_attrs, steerable_edge_attrs,
           W_embed, b_embed, We1_0, be1_0, We2_0, be2_0, Wn_0, bn_0,
           We1_1, be1_1, We2_1, be2_1, Wn_1, bn_1,
           We1_2, be1_2, We2_2, be2_2, Wn_2, bn_2):
    src = edge_index[0]
    dst = edge_index[1]
    pad = EPAD - E
    zi = jnp.zeros((pad,), jnp.int32)
    srcm = jnp.concatenate([src, zi]).reshape(EPAD // IDXW, IDXW)
    dstg = jnp.concatenate([dst, zi]).reshape(EPAD // IDXW, IDXW)
    dstp = jnp.concatenate([dst, jnp.full((pad,), DUMMY, jnp.int32)])
    dsts = (dstp >> 1).reshape(EPAD // IDXW, IDXW)
    parity = (dstp & 1).astype(jnp.float32).reshape(EPAD, 1)
    eap = jnp.concatenate(
        [jnp.concatenate(
            [steerable_edge_attrs, jnp.zeros((pad, DA), jnp.float32)], axis=0),
         parity, jnp.zeros((EPAD, 3), jnp.float32)], axis=1)
    na = steerable_node_attrs

    nodes = _tc_embed(x, na, _rw(W_embed), b_embed.reshape(1, H))
    for (We1, be1, We2, be2, Wn, bn) in (
            (We1_0, be1_0, We2_0, be2_0, Wn_0, bn_0),
            (We1_1, be1_1, We2_1, be2_1, Wn_1, bn_1),
            (We1_2, be1_2, We2_2, be2_2, Wn_2, bn_2)):
        W1 = _rw(We1)
        W2 = _rw(We2)
        Wn_r = _rw(Wn)
        b1 = be1.reshape(1, H)
        b2 = be2.reshape(1, H)
        gsA, gdA = _sc_gather_a(nodes, srcm, dstg)
        gsB, gdB = _sc_gather_b(nodes, srcm, dstg)
        m2A = _tc_edge_a(gsA, gdA, eap, W1[:H], W1[H:], b1, W2, b2)
        m2B = _tc_edge_b(gsB, gdB, eap, W1[:H], W1[H:], b1, W2, b2)
        aggp = _sc_scatter(m2A, m2B, dsts).reshape(NC, NPAD, H)
        nodes = _tc_node(nodes, aggp, na, Wn_r[:H], Wn_r[H:],
                         bn.reshape(1, H))
    return nodes[:, :H]
